# baseline (speedup 1.0000x reference)
import functools

import jax
import jax.numpy as jnp
from jax import lax
from jax.experimental import pallas as pl
from jax.experimental.pallas import tpu as pltpu
from jax.experimental.pallas import tpu_sc as plsc

NC = 2
NS = 16
LANES = 16
NW = NC * NS


def _rsqrt(x):
    i = plsc.bitcast(x, jnp.int32)
    i = jnp.int32(0x5F3759DF) - (i >> 1)
    y = plsc.bitcast(i, jnp.float32)
    for _ in range(3):
        y = y * (1.5 - 0.5 * x * y * y)
    return y


def _gather_normalize(table, idx2d, n_rows):
    D = table.shape[1]
    per_w = n_rows // NW
    CH = 1280
    SUB = 128
    n_sub = CH // SUB
    n_chunks = per_w // CH
    n_pairs = n_chunks // 2
    groups = CH // LANES

    mesh = plsc.VectorSubcoreMesh(core_axis_name="c", subcore_axis_name="s")

    @functools.partial(
        pl.kernel,
        out_type=jax.ShapeDtypeStruct((n_rows, D), jnp.float32),
        mesh=mesh,
        compiler_params=pltpu.CompilerParams(
            needs_layout_passes=False, use_tc_tiling_on_sc=False),
        scratch_types=[
            pltpu.VMEM((n_sub, SUB), jnp.int32),
            pltpu.VMEM((n_sub, SUB), jnp.int32),
            pltpu.VMEM((CH, D), jnp.float32),
            pltpu.VMEM((CH, D), jnp.float32),
            pltpu.SemaphoreType.DMA,
            pltpu.SemaphoreType.DMA,
            pltpu.SemaphoreType.DMA,
            pltpu.SemaphoreType.DMA,
        ],
    )
    def k(table_hbm, idx_hbm, out_hbm, i0, i1, d0, d1, sga, sgb, so0, so1):
        wid = lax.axis_index("s") * NC + lax.axis_index("c")
        wbase = wid * per_w
        wrow = wid * (per_w // SUB)

        def fire(c, ibuf, dbuf, sem):
            pltpu.sync_copy(idx_hbm.at[pl.ds(wrow + c * n_sub, n_sub)], ibuf)
            for j in range(n_sub):
                pltpu.async_copy(
                    table_hbm.at[ibuf.at[j]],
                    dbuf.at[pl.ds(j * SUB, SUB)],
                    sem,
                )

        def drain_gathers(ibuf, dbuf, sem):
            for j in range(n_sub):
                pltpu.make_async_copy(
                    table_hbm.at[ibuf.at[j]],
                    dbuf.at[pl.ds(j * SUB, SUB)],
                    sem,
                ).wait()

        def normalize(dbuf):
            @pl.loop(0, groups)
            def _group(grp):
                r0 = grp * LANES
                lane = lax.iota(jnp.int32, LANES)
                rvec = r0 + lane
                cols = []
                acc = jnp.zeros((LANES,), jnp.float32)
                diag = [(lane + e) & (D - 1) for e in range(D)]
                for e in range(D):
                    v = plsc.load_gather(dbuf, [rvec, diag[e]])
                    cols.append(v)
                    acc = acc + v * v
                y = _rsqrt(acc)
                for e in range(D):
                    plsc.store_scatter(dbuf, [rvec, diag[e]], cols[e] * y)

        def finish(c, ibuf, dbuf, semg, semo):
            drain_gathers(ibuf, dbuf, semg)
            normalize(dbuf)
            pltpu.async_copy(dbuf, out_hbm.at[pl.ds(wbase + c * CH, CH)], semo)

        def drain_out(c, dbuf, semo):
            pltpu.make_async_copy(
                dbuf, out_hbm.at[pl.ds(wbase + c * CH, CH)], semo).wait()

        fire(0, i0, d0, sga)

        @pl.loop(0, n_pairs)
        def _pair(p):
            c0 = 2 * p
            c1 = c0 + 1

            @pl.when(p > 0)
            def _():
                drain_out(c1 - 2, d1, so1)
            fire(c1, i1, d1, sgb)
            finish(c0, i0, d0, sga, so0)

            @pl.when(p < n_pairs - 1)
            def _():
                drain_out(c0, d0, so0)
                fire(c0 + 2, i0, d0, sga)
            finish(c1, i1, d1, sgb, so1)

        drain_out(n_chunks - 2, d0, so0)
        drain_out(n_chunks - 1, d1, so1)

    return k(table, idx2d)


def kernel(features, table):
    B, S = features.shape
    D = table.shape[1]
    n = B * S
    idx2d = features.reshape(n // 128, 128)
    out = _gather_normalize(table, idx2d, n)
    return out.reshape(B, S, D)


# speedup vs baseline: 2.1006x; 2.1006x over previous
import functools

import jax
import jax.numpy as jnp
from jax import lax
from jax.experimental import pallas as pl
from jax.experimental.pallas import tpu as pltpu
from jax.experimental.pallas import tpu_sc as plsc

NC = 2
NS = 16
LANES = 16
NW = NC * NS


def _rsqrt(x):
    i = plsc.bitcast(x, jnp.int32)
    i = jnp.int32(0x5F3759DF) - (i >> 1)
    y = plsc.bitcast(i, jnp.float32)
    for _ in range(3):
        y = y * (1.5 - 0.5 * x * y * y)
    return y


def _gather_normalize(table, idx2d, batch, seq):
    D = table.shape[1]
    NB = LANES
    CH = NB * seq
    SUB = 80
    n_sub = CH // SUB
    b_per_w = batch // NW
    n_chunks = b_per_w // NB
    n_pairs = n_chunks // 2

    mesh = plsc.VectorSubcoreMesh(core_axis_name="c", subcore_axis_name="s")

    @functools.partial(
        pl.kernel,
        out_type=jax.ShapeDtypeStruct((seq, D, batch), jnp.float32),
        mesh=mesh,
        compiler_params=pltpu.CompilerParams(
            needs_layout_passes=False, use_tc_tiling_on_sc=False),
        scratch_types=[
            pltpu.VMEM((n_sub, SUB), jnp.int32),
            pltpu.VMEM((n_sub, SUB), jnp.int32),
            pltpu.VMEM((CH, D), jnp.float32),
            pltpu.VMEM((CH, D), jnp.float32),
            pltpu.VMEM((seq, D, NB), jnp.float32),
            pltpu.VMEM((seq, D, NB), jnp.float32),
            pltpu.SemaphoreType.DMA,
            pltpu.SemaphoreType.DMA,
            pltpu.SemaphoreType.DMA,
            pltpu.SemaphoreType.DMA,
        ],
    )
    def k(table_hbm, idx_hbm, out_hbm,
          i0, i1, d0, d1, t0, t1, sga, sgb, so0, so1):
        wid = lax.axis_index("s") * NC + lax.axis_index("c")
        wb = wid * b_per_w
        wrow = wid * (b_per_w * seq // SUB)

        def fire(c, ibuf, dbuf, sem):
            pltpu.sync_copy(idx_hbm.at[pl.ds(wrow + c * n_sub, n_sub)], ibuf)
            for j in range(n_sub):
                pltpu.async_copy(
                    table_hbm.at[ibuf.at[j]],
                    dbuf.at[pl.ds(j * SUB, SUB)],
                    sem,
                )

        def drain_gathers(ibuf, dbuf, sem):
            for j in range(n_sub):
                pltpu.make_async_copy(
                    table_hbm.at[ibuf.at[j]],
                    dbuf.at[pl.ds(j * SUB, SUB)],
                    sem,
                ).wait()

        def normalize(dbuf, tbuf):
            @pl.loop(0, seq)
            def _step(s):
                lane = lax.iota(jnp.int32, LANES)
                rvec = s + seq * lane
                svec = jnp.full((LANES,), s, jnp.int32)
                diag = [(lane + e) & (D - 1) for e in range(D)]
                cols = []
                acc = jnp.zeros((LANES,), jnp.float32)
                for e in range(D):
                    v = plsc.load_gather(dbuf, [rvec, diag[e]])
                    cols.append(v)
                    acc = acc + v * v
                y = _rsqrt(acc)
                for e in range(D):
                    plsc.store_scatter(tbuf, [svec, diag[e], lane],
                                       cols[e] * y)

        def finish(c, ibuf, dbuf, tbuf, semg, semo):
            drain_gathers(ibuf, dbuf, semg)
            normalize(dbuf, tbuf)
            pltpu.async_copy(
                tbuf, out_hbm.at[:, :, pl.ds(wb + c * NB, NB)], semo)

        def drain_out(c, tbuf, semo):
            pltpu.make_async_copy(
                tbuf, out_hbm.at[:, :, pl.ds(wb + c * NB, NB)], semo).wait()

        fire(0, i0, d0, sga)

        @pl.loop(0, n_pairs)
        def _pair(p):
            c0 = 2 * p
            c1 = c0 + 1

            @pl.when(p > 0)
            def _():
                drain_out(c1 - 2, t1, so1)
            fire(c1, i1, d1, sgb)
            finish(c0, i0, d0, t0, sga, so0)

            @pl.when(p < n_pairs - 1)
            def _():
                drain_out(c0, t0, so0)
                fire(c0 + 2, i0, d0, sga)
            finish(c1, i1, d1, t1, sgb, so1)

        drain_out(n_chunks - 2, t0, so0)
        drain_out(n_chunks - 1, t1, so1)

    return k(table, idx2d)


def kernel(features, table):
    B, S = features.shape
    D = table.shape[1]
    idx2d = features.reshape(B * S // 80, 80)
    res = _gather_normalize(table, idx2d, B, S)
    return jnp.transpose(res, (2, 0, 1))
